# Initial kernel scaffold; baseline (speedup 1.0000x reference)
import jax
import jax.numpy as jnp
from jax.experimental import pallas as pl


def kernel(x, mapping, edge_index, edge_attr, batch, emb, W1, b1, W2, b2):
    raise NotImplementedError("write your pallas kernel here")


# speedup vs baseline: 9.1004x; 9.1004x over previous
import jax
import jax.numpy as jnp
from jax import lax
from jax.experimental import pallas as pl
from jax.experimental.pallas import tpu as pltpu
from jax.experimental.pallas import tpu_sc as plsc

NN = 10000
NNP = 10240
NE = 320000
FEAT = 128
EMB = 64
HID = 128
NG = 32
EW = 128
N_EWIN = NE // EW
GW = 128
N_GWIN = NNP // GW
NC = 2
NS = 16
NWORK = NC * NS
DPT = NNP // NS

_HIGH = lax.Precision.HIGHEST


def _f32(shape):
    return jax.ShapeDtypeStruct(shape, jnp.float32)


def _sc_prep_body(map_hbm, dst_hbm, attr_hbm, emb2_hbm,
                  nemb_hbm, deg_hbm,
                  mapv, mapv2, rows, dsti, attrv, zbuf, deg_sh, sem):
    c = lax.axis_index("c")
    s = lax.axis_index("s")
    wid = c * NS + s

    z16 = jnp.zeros((16,), jnp.float32)

    def zb(i, carry):
        zbuf[pl.ds(i * 16, 16)] = z16
        return carry

    lax.fori_loop(0, DPT // 16, zb, 0)
    pltpu.sync_copy(zbuf, deg_sh.at[pl.ds(s * DPT, DPT)])
    plsc.subcore_barrier()

    n_e = (N_EWIN - wid + NWORK - 1) // NWORK

    def ebody(k, carry):
        base = (wid + k * NWORK) * EW
        pltpu.sync_copy(dst_hbm.at[pl.ds(base, EW)], dsti.at[0])
        pltpu.sync_copy(attr_hbm.at[pl.ds(base, EW)], attrv)
        pltpu.sync_copy(attrv, deg_sh.at[dsti.at[0]], add=True)
        return carry

    lax.fori_loop(0, n_e, ebody, 0)

    n_g = (N_GWIN - wid + NWORK - 1) // NWORK

    def gbody(k, carry):
        base = (wid + k * NWORK) * GW
        pltpu.sync_copy(map_hbm.at[pl.ds(base, GW)], mapv)
        for j in range(GW // 16):
            sl = pl.ds(j * 16, 16)
            mapv2[sl] = mapv[sl] >> 1
        pltpu.async_copy(emb2_hbm.at[mapv2], rows, sem).wait()
        pltpu.sync_copy(rows, nemb_hbm.at[pl.ds(base, GW)])
        return carry

    lax.fori_loop(0, n_g, gbody, 0)

    plsc.subcore_barrier()

    pltpu.sync_copy(deg_sh.at[pl.ds(s * DPT, DPT)], zbuf)
    pltpu.sync_copy(zbuf, deg_hbm.at[pl.ds(c * NNP + s * DPT, DPT)])


def _sc_prep(map_pad, dst, attr, emb2):
    mesh = plsc.VectorSubcoreMesh(core_axis_name="c", subcore_axis_name="s")
    f = pl.kernel(
        _sc_prep_body,
        out_type=(_f32((NNP, FEAT)), _f32((2 * NNP,))),
        mesh=mesh,
        scratch_types=[
            pltpu.VMEM((GW,), jnp.int32),
            pltpu.VMEM((GW,), jnp.int32),
            pltpu.VMEM((GW, FEAT), jnp.float32),
            pltpu.VMEM((1, EW), jnp.int32),
            pltpu.VMEM((EW,), jnp.float32),
            pltpu.VMEM((DPT,), jnp.float32),
            pltpu.VMEM_SHARED((NNP,), jnp.float32),
            pltpu.SemaphoreType.DMA,
        ],
        compiler_params=pltpu.CompilerParams(needs_layout_passes=False),
    )
    return f(map_pad, dst, attr, emb2)


def _sc_mp_body(y_hbm, src_hbm, dst_hbm, attr_hbm,
                out_hbm,
                srci, dsti, wv, rows, acc_sh, sem):
    c = lax.axis_index("c")
    s = lax.axis_index("s")
    wid = c * NS + s
    lane = lax.iota(jnp.int32, 16)

    z16 = jnp.zeros((16,), jnp.float32)

    def zrow(i, carry):
        for j in range(HID // 16):
            rows[i, pl.ds(j * 16, 16)] = z16
        return carry

    lax.fori_loop(0, EW, zrow, 0)
    for cb in range(0, DPT, EW):
        pltpu.sync_copy(rows, acc_sh.at[pl.ds(s * DPT + cb, EW)])
    plsc.subcore_barrier()

    n_e = (N_EWIN - wid + NWORK - 1) // NWORK

    def ebody(k, carry):
        base = (wid + k * NWORK) * EW
        pltpu.sync_copy(src_hbm.at[pl.ds(base, EW)], srci)
        pltpu.sync_copy(dst_hbm.at[pl.ds(base, EW)], dsti.at[0])
        pltpu.sync_copy(attr_hbm.at[pl.ds(base, EW)], wv)
        pltpu.async_copy(y_hbm.at[srci], rows, sem).wait()

        def scale(i, cc):
            ri = jnp.full((16,), i, dtype=jnp.int32)
            ws = plsc.load_gather(wv, [ri])
            for j in range(HID // 16):
                sl = pl.ds(j * 16, 16)
                rows[i, sl] = rows[i, sl] * ws
            return cc

        lax.fori_loop(0, EW, scale, 0)
        pltpu.sync_copy(rows, acc_sh.at[dsti.at[0]], add=True)
        return carry

    lax.fori_loop(0, n_e, ebody, 0)

    plsc.subcore_barrier()
    for cb in range(0, DPT, EW):
        pltpu.sync_copy(acc_sh.at[pl.ds(s * DPT + cb, EW)], rows)
        pltpu.sync_copy(rows, out_hbm.at[pl.ds(c * NNP + s * DPT + cb, EW)])


def _sc_mp(y, src, dst, attr):
    mesh = plsc.VectorSubcoreMesh(core_axis_name="c", subcore_axis_name="s")
    f = pl.kernel(
        _sc_mp_body,
        out_type=_f32((2 * NNP, HID)),
        mesh=mesh,
        scratch_types=[
            pltpu.VMEM((EW,), jnp.int32),
            pltpu.VMEM((1, EW), jnp.int32),
            pltpu.VMEM((EW,), jnp.float32),
            pltpu.VMEM((EW, HID), jnp.float32),
            pltpu.VMEM_SHARED((NNP, HID), jnp.float32),
            pltpu.SemaphoreType.DMA,
        ],
        compiler_params=pltpu.CompilerParams(needs_layout_passes=False),
    )
    return f(y, src, dst, attr)


_RB = 1000
_NRB = NN // _RB


def _dis(da, db):
    deg = da + db + 1.0
    return jnp.where(deg > 0, lax.rsqrt(jnp.maximum(deg, 1e-12)), 0.0)


def _mm(a, b):
    return lax.dot_general(a, b, (((1,), (0,)), ((), ())),
                           precision=_HIGH, preferred_element_type=jnp.float32)


def _mm1_body(x_ref, ne_ref, m_ref, da_ref, db_ref, w1x_ref, w1e_ref, y_ref):
    dis = _dis(da_ref[...], db_ref[...])
    par = (m_ref[...] & 1) == 1
    ne = ne_ref[...]
    e = jnp.where(par, ne[:, EMB:], ne[:, :EMB])
    xw = _mm(x_ref[...], w1x_ref[...]) + _mm(e, w1e_ref[...])
    y_ref[...] = dis * xw


def _tc_mm1(x, nemb2, map_col, dega, degb, w1x, w1e):
    return pl.pallas_call(
        _mm1_body,
        grid=(_NRB,),
        in_specs=[
            pl.BlockSpec((_RB, FEAT), lambda i: (i, 0)),
            pl.BlockSpec((_RB, FEAT), lambda i: (i, 0)),
            pl.BlockSpec((_RB, 1), lambda i: (i, 0)),
            pl.BlockSpec((_RB, 1), lambda i: (i, 0)),
            pl.BlockSpec((_RB, 1), lambda i: (i, 0)),
            pl.BlockSpec((FEAT, HID), lambda i: (0, 0)),
            pl.BlockSpec((EMB, HID), lambda i: (0, 0)),
        ],
        out_specs=pl.BlockSpec((_RB, HID), lambda i: (i, 0)),
        out_shape=_f32((NN, HID)),
    )(x, nemb2, map_col, dega, degb, w1x, w1e)


def _mm2_body(aa_ref, ab_ref, y1_ref, da_ref, db_ref, b1_ref, w2_ref, y2_ref):
    dis = _dis(da_ref[...], db_ref[...])
    h1 = jnp.maximum(dis * (aa_ref[...] + ab_ref[...] + y1_ref[...]) + b1_ref[...], 0.0)
    y2_ref[...] = dis * _mm(h1, w2_ref[...])


def _tc_mm2(acca, accb, y1, dega, degb, b1, w2):
    return pl.pallas_call(
        _mm2_body,
        grid=(_NRB,),
        in_specs=[
            pl.BlockSpec((_RB, HID), lambda i: (i, 0)),
            pl.BlockSpec((_RB, HID), lambda i: (i, 0)),
            pl.BlockSpec((_RB, HID), lambda i: (i, 0)),
            pl.BlockSpec((_RB, 1), lambda i: (i, 0)),
            pl.BlockSpec((_RB, 1), lambda i: (i, 0)),
            pl.BlockSpec((1, HID), lambda i: (0, 0)),
            pl.BlockSpec((HID, HID), lambda i: (0, 0)),
        ],
        out_specs=pl.BlockSpec((_RB, HID), lambda i: (i, 0)),
        out_shape=_f32((NN, HID)),
    )(acca, accb, y1, dega, degb, b1, w2)


def _pool_body(aa_ref, ab_ref, y2_ref, da_ref, db_ref, b2_ref, batch_ref,
               out_ref, cnt_ref):
    i = pl.program_id(0)

    @pl.when(i == 0)
    def _():
        out_ref[...] = jnp.zeros_like(out_ref)
        cnt_ref[...] = jnp.zeros_like(cnt_ref)

    dis = _dis(da_ref[...], db_ref[...])
    h2 = jnp.maximum(dis * (aa_ref[...] + ab_ref[...] + y2_ref[...]) + b2_ref[...], 0.0)
    b = batch_ref[0]
    gi = lax.broadcasted_iota(jnp.int32, (NG, _RB), 0)
    onehot = (gi == jnp.broadcast_to(b, (NG, _RB))).astype(jnp.float32)
    out_ref[...] += _mm(onehot, h2)
    cnt_ref[...] += jnp.broadcast_to(
        jnp.sum(onehot, axis=1, keepdims=True), (NG, HID))

    @pl.when(i == _NRB - 1)
    def _():
        out_ref[...] = out_ref[...] / jnp.maximum(cnt_ref[...], 1.0)


def _tc_pool(acca, accb, y2, dega, degb, b2, batch3):
    return pl.pallas_call(
        _pool_body,
        grid=(_NRB,),
        in_specs=[
            pl.BlockSpec((_RB, HID), lambda i: (i, 0)),
            pl.BlockSpec((_RB, HID), lambda i: (i, 0)),
            pl.BlockSpec((_RB, HID), lambda i: (i, 0)),
            pl.BlockSpec((_RB, 1), lambda i: (i, 0)),
            pl.BlockSpec((_RB, 1), lambda i: (i, 0)),
            pl.BlockSpec((1, HID), lambda i: (0, 0)),
            pl.BlockSpec((1, 1, _RB), lambda i: (i, 0, 0)),
        ],
        out_specs=pl.BlockSpec((NG, HID), lambda i: (0, 0)),
        out_shape=_f32((NG, HID)),
        scratch_shapes=[pltpu.VMEM((NG, HID), jnp.float32)],
    )(acca, accb, y2, dega, degb, b2, batch3)


def kernel(x, mapping, edge_index, edge_attr, batch, emb, W1, b1, W2, b2):
    src = edge_index[0]
    dst = edge_index[1]
    map_pad = jnp.concatenate([mapping, jnp.zeros((NNP - NN,), jnp.int32)])
    emb2 = emb.reshape(emb.shape[0] // 2, 2 * EMB)

    nemb2, deg = _sc_prep(map_pad, dst, edge_attr, emb2)
    dega = deg[:NN].reshape(NN, 1)
    degb = deg[NNP:NNP + NN].reshape(NN, 1)

    y1 = _tc_mm1(x, nemb2[:NN], mapping.reshape(NN, 1), dega, degb,
                 W1[:FEAT], W1[FEAT:])

    acc1 = _sc_mp(y1, src, dst, edge_attr)
    y2 = _tc_mm2(acc1[:NN], acc1[NNP:NNP + NN], y1, dega, degb,
                 b1.reshape(1, HID), W2)

    acc2 = _sc_mp(y2, src, dst, edge_attr)
    out = _tc_pool(acc2[:NN], acc2[NNP:NNP + NN], y2, dega, degb,
                   b2.reshape(1, HID), batch.reshape(_NRB, 1, _RB))
    return out


# speedup vs baseline: 14.3184x; 1.5734x over previous
import jax
import jax.numpy as jnp
from jax import lax
from jax.experimental import pallas as pl
from jax.experimental.pallas import tpu as pltpu
from jax.experimental.pallas import tpu_sc as plsc

NN = 10000
NNP = 10240
NE = 320000
FEAT = 128
EMB = 64
HID = 128
NG = 32
EW = 128
N_EWIN = NE // EW
GW = 128
N_GWIN = NNP // GW
NC = 2
NS = 16
NWORK = NC * NS
DPT = NNP // NS

_HIGH = lax.Precision.HIGHEST


def _f32(shape):
    return jax.ShapeDtypeStruct(shape, jnp.float32)


def _sc_prep_body(map_hbm, dst_hbm, attr_hbm, emb2_hbm,
                  nemb_hbm, deg_hbm,
                  mapv, mapv2, rows, dsti, attrv, zbuf, deg_sh, sem):
    c = lax.axis_index("c")
    s = lax.axis_index("s")
    wid = c * NS + s

    z16 = jnp.zeros((16,), jnp.float32)

    def zb(i, carry):
        zbuf[pl.ds(i * 16, 16)] = z16
        return carry

    lax.fori_loop(0, DPT // 16, zb, 0)
    pltpu.sync_copy(zbuf, deg_sh.at[pl.ds(s * DPT, DPT)])
    plsc.subcore_barrier()

    n_e = (N_EWIN - wid + NWORK - 1) // NWORK

    def ebody(k, carry):
        base = (wid + k * NWORK) * EW
        pltpu.sync_copy(dst_hbm.at[pl.ds(base, EW)], dsti.at[0])
        pltpu.sync_copy(attr_hbm.at[pl.ds(base, EW)], attrv)
        pltpu.sync_copy(attrv, deg_sh.at[dsti.at[0]], add=True)
        return carry

    lax.fori_loop(0, n_e, ebody, 0)

    n_g = (N_GWIN - wid + NWORK - 1) // NWORK

    def gbody(k, carry):
        base = (wid + k * NWORK) * GW
        pltpu.sync_copy(map_hbm.at[pl.ds(base, GW)], mapv)
        for j in range(GW // 16):
            sl = pl.ds(j * 16, 16)
            mapv2[sl] = mapv[sl] >> 1
        pltpu.async_copy(emb2_hbm.at[mapv2], rows, sem).wait()
        pltpu.sync_copy(rows, nemb_hbm.at[pl.ds(base, GW)])
        return carry

    lax.fori_loop(0, n_g, gbody, 0)

    plsc.subcore_barrier()

    pltpu.sync_copy(deg_sh.at[pl.ds(s * DPT, DPT)], zbuf)
    pltpu.sync_copy(zbuf, deg_hbm.at[pl.ds(c * NNP + s * DPT, DPT)])


def _sc_prep(map_pad, dst, attr, emb2):
    mesh = plsc.VectorSubcoreMesh(core_axis_name="c", subcore_axis_name="s")
    f = pl.kernel(
        _sc_prep_body,
        out_type=(_f32((NNP, FEAT)), _f32((2 * NNP,))),
        mesh=mesh,
        scratch_types=[
            pltpu.VMEM((GW,), jnp.int32),
            pltpu.VMEM((GW,), jnp.int32),
            pltpu.VMEM((GW, FEAT), jnp.float32),
            pltpu.VMEM((1, EW), jnp.int32),
            pltpu.VMEM((EW,), jnp.float32),
            pltpu.VMEM((DPT,), jnp.float32),
            pltpu.VMEM_SHARED((NNP,), jnp.float32),
            pltpu.SemaphoreType.DMA,
        ],
        compiler_params=pltpu.CompilerParams(needs_layout_passes=False),
    )
    return f(map_pad, dst, attr, emb2)


MW = 64
N_MWIN = NE // MW
WPW = 156


def _sc_mp_body(y_hbm, src_hbm, dst_hbm, attr_hbm,
                out_hbm,
                ra, rb, rc, sia, sib, sic, dxa, dxb, dxc, wva, wvb, wvc,
                acc_sh,
                sga, sgb, sgc, ssa, ssb, ssc, sla, slb, slc):
    c = lax.axis_index("c")
    s = lax.axis_index("s")
    wid = c * NS + s
    z16 = jnp.zeros((16,), jnp.float32)
    bufs = ((ra, sia, dxa, wva, sga, ssa, sla),
            (rb, sib, dxb, wvb, sgb, ssb, slb),
            (rc, sic, dxc, wvc, sgc, ssc, slc))
    wbase = wid * WPW

    def zrow(i, carry):
        for j in range(HID // 16):
            ra[i, pl.ds(j * 16, 16)] = z16
        return carry

    lax.fori_loop(0, MW, zrow, 0)
    for cb in range(0, DPT, MW):
        pltpu.sync_copy(ra, acc_sh.at[pl.ds(s * DPT + cb, MW)])
    plsc.subcore_barrier()

    def scale(b):
        rows, wv = bufs[b][0], bufs[b][3]

        def body(i, cc):
            ws = plsc.load_gather(wv, [jnp.full((16,), i, dtype=jnp.int32)])
            for j in range(HID // 16):
                sl = pl.ds(j * 16, 16)
                rows[i, sl] = rows[i, sl] * ws
            return cc

        lax.fori_loop(0, MW, body, 0)

    def start_idx_loads(b, k):
        _, si, dx, wv, _, _, sl = bufs[b]
        ebase = (wbase + k) * MW
        pltpu.async_copy(src_hbm.at[pl.ds(ebase, MW)], si, sl)
        pltpu.async_copy(dst_hbm.at[pl.ds(ebase, MW)], dx.at[0], sl)
        pltpu.async_copy(attr_hbm.at[pl.ds(ebase, MW)], wv, sl)

    def wait_idx_loads(b, k):
        _, si, dx, wv, _, _, sl = bufs[b]
        ebase = (wbase + k) * MW
        pltpu.make_async_copy(src_hbm.at[pl.ds(ebase, MW)], si, sl).wait()
        pltpu.make_async_copy(dst_hbm.at[pl.ds(ebase, MW)], dx.at[0], sl).wait()
        pltpu.make_async_copy(attr_hbm.at[pl.ds(ebase, MW)], wv, sl).wait()

    def start_gather(b):
        rows, si, sg = bufs[b][0], bufs[b][1], bufs[b][4]
        pltpu.async_copy(y_hbm.at[si], rows, sg)

    def wait_gather(b):
        rows, si, sg = bufs[b][0], bufs[b][1], bufs[b][4]
        pltpu.make_async_copy(y_hbm.at[si], rows, sg).wait()

    def start_scatter(b):
        rows, dx, ss = bufs[b][0], bufs[b][2], bufs[b][5]
        pltpu.async_copy(rows, acc_sh.at[dx.at[0]], ss, add=True)

    def wait_scatter(b):
        rows, dx, ss = bufs[b][0], bufs[b][2], bufs[b][5]
        pltpu.make_async_copy(rows, acc_sh.at[dx.at[0]], ss).wait()

    start_idx_loads(0, 0)
    start_idx_loads(1, 1)
    wait_idx_loads(0, 0)
    start_gather(0)

    def pbody(p, carry):
        for j in range(3):
            k = 3 * p + j
            b, bB, bA = j, (j + 1) % 3, (j + 2) % 3

            @pl.when(k + 1 < WPW)
            def _():
                wait_idx_loads(bB, k + 1)
                start_gather(bB)

            wait_gather(b)
            scale(b)
            start_scatter(b)

            @pl.when(k + 2 < WPW)
            def _():
                @pl.when(k >= 1)
                def _():
                    wait_scatter(bA)

                start_idx_loads(bA, k + 2)
        return carry

    lax.fori_loop(0, WPW // 3, pbody, 0)
    for b in range(3):
        wait_scatter(b)

    @pl.when(wid < N_MWIN - NWORK * WPW)
    def _():
        lbase = (NWORK * WPW + wid) * MW
        pltpu.sync_copy(src_hbm.at[pl.ds(lbase, MW)], sia)
        pltpu.sync_copy(dst_hbm.at[pl.ds(lbase, MW)], dxa.at[0])
        pltpu.sync_copy(attr_hbm.at[pl.ds(lbase, MW)], wva)
        start_gather(0)
        wait_gather(0)
        scale(0)
        pltpu.sync_copy(ra, acc_sh.at[dxa.at[0]], add=True)

    plsc.subcore_barrier()
    for cb in range(0, DPT, MW):
        pltpu.sync_copy(acc_sh.at[pl.ds(s * DPT + cb, MW)], ra)
        pltpu.sync_copy(ra, out_hbm.at[pl.ds(c * NNP + s * DPT + cb, MW)])


def _sc_mp(y, src, dst, attr):
    mesh = plsc.VectorSubcoreMesh(core_axis_name="c", subcore_axis_name="s")
    f = pl.kernel(
        _sc_mp_body,
        out_type=_f32((2 * NNP, HID)),
        mesh=mesh,
        scratch_types=[
            pltpu.VMEM((MW, HID), jnp.float32),
            pltpu.VMEM((MW, HID), jnp.float32),
            pltpu.VMEM((MW, HID), jnp.float32),
            pltpu.VMEM((MW,), jnp.int32),
            pltpu.VMEM((MW,), jnp.int32),
            pltpu.VMEM((MW,), jnp.int32),
            pltpu.VMEM((1, MW), jnp.int32),
            pltpu.VMEM((1, MW), jnp.int32),
            pltpu.VMEM((1, MW), jnp.int32),
            pltpu.VMEM((MW,), jnp.float32),
            pltpu.VMEM((MW,), jnp.float32),
            pltpu.VMEM((MW,), jnp.float32),
            pltpu.VMEM_SHARED((NNP, HID), jnp.float32),
            pltpu.SemaphoreType.DMA,
            pltpu.SemaphoreType.DMA,
            pltpu.SemaphoreType.DMA,
            pltpu.SemaphoreType.DMA,
            pltpu.SemaphoreType.DMA,
            pltpu.SemaphoreType.DMA,
            pltpu.SemaphoreType.DMA,
            pltpu.SemaphoreType.DMA,
            pltpu.SemaphoreType.DMA,
        ],
        compiler_params=pltpu.CompilerParams(needs_layout_passes=False),
    )
    return f(y, src, dst, attr)


_RB = 1000
_NRB = NN // _RB


def _dis(da, db):
    deg = da + db + 1.0
    return jnp.where(deg > 0, lax.rsqrt(jnp.maximum(deg, 1e-12)), 0.0)


def _mm(a, b):
    return lax.dot_general(a, b, (((1,), (0,)), ((), ())),
                           precision=_HIGH, preferred_element_type=jnp.float32)


def _mm1_body(x_ref, ne_ref, m_ref, da_ref, db_ref, w1x_ref, w1e_ref, y_ref):
    dis = _dis(da_ref[...], db_ref[...])
    par = (m_ref[...] & 1) == 1
    ne = ne_ref[...]
    e = jnp.where(par, ne[:, EMB:], ne[:, :EMB])
    xw = _mm(x_ref[...], w1x_ref[...]) + _mm(e, w1e_ref[...])
    y_ref[...] = dis * xw


def _tc_mm1(x, nemb2, map_col, dega, degb, w1x, w1e):
    return pl.pallas_call(
        _mm1_body,
        grid=(_NRB,),
        in_specs=[
            pl.BlockSpec((_RB, FEAT), lambda i: (i, 0)),
            pl.BlockSpec((_RB, FEAT), lambda i: (i, 0)),
            pl.BlockSpec((_RB, 1), lambda i: (i, 0)),
            pl.BlockSpec((_RB, 1), lambda i: (i, 0)),
            pl.BlockSpec((_RB, 1), lambda i: (i, 0)),
            pl.BlockSpec((FEAT, HID), lambda i: (0, 0)),
            pl.BlockSpec((EMB, HID), lambda i: (0, 0)),
        ],
        out_specs=pl.BlockSpec((_RB, HID), lambda i: (i, 0)),
        out_shape=_f32((NN, HID)),
    )(x, nemb2, map_col, dega, degb, w1x, w1e)


def _mm2_body(aa_ref, ab_ref, y1_ref, da_ref, db_ref, b1_ref, w2_ref, y2_ref):
    dis = _dis(da_ref[...], db_ref[...])
    h1 = jnp.maximum(dis * (aa_ref[...] + ab_ref[...] + y1_ref[...]) + b1_ref[...], 0.0)
    y2_ref[...] = dis * _mm(h1, w2_ref[...])


def _tc_mm2(acca, accb, y1, dega, degb, b1, w2):
    return pl.pallas_call(
        _mm2_body,
        grid=(_NRB,),
        in_specs=[
            pl.BlockSpec((_RB, HID), lambda i: (i, 0)),
            pl.BlockSpec((_RB, HID), lambda i: (i, 0)),
            pl.BlockSpec((_RB, HID), lambda i: (i, 0)),
            pl.BlockSpec((_RB, 1), lambda i: (i, 0)),
            pl.BlockSpec((_RB, 1), lambda i: (i, 0)),
            pl.BlockSpec((1, HID), lambda i: (0, 0)),
            pl.BlockSpec((HID, HID), lambda i: (0, 0)),
        ],
        out_specs=pl.BlockSpec((_RB, HID), lambda i: (i, 0)),
        out_shape=_f32((NN, HID)),
    )(acca, accb, y1, dega, degb, b1, w2)


def _pool_body(aa_ref, ab_ref, y2_ref, da_ref, db_ref, b2_ref, batch_ref,
               out_ref, cnt_ref):
    i = pl.program_id(0)

    @pl.when(i == 0)
    def _():
        out_ref[...] = jnp.zeros_like(out_ref)
        cnt_ref[...] = jnp.zeros_like(cnt_ref)

    dis = _dis(da_ref[...], db_ref[...])
    h2 = jnp.maximum(dis * (aa_ref[...] + ab_ref[...] + y2_ref[...]) + b2_ref[...], 0.0)
    b = batch_ref[0]
    gi = lax.broadcasted_iota(jnp.int32, (NG, _RB), 0)
    onehot = (gi == jnp.broadcast_to(b, (NG, _RB))).astype(jnp.float32)
    out_ref[...] += _mm(onehot, h2)
    cnt_ref[...] += jnp.broadcast_to(
        jnp.sum(onehot, axis=1, keepdims=True), (NG, HID))

    @pl.when(i == _NRB - 1)
    def _():
        out_ref[...] = out_ref[...] / jnp.maximum(cnt_ref[...], 1.0)


def _tc_pool(acca, accb, y2, dega, degb, b2, batch3):
    return pl.pallas_call(
        _pool_body,
        grid=(_NRB,),
        in_specs=[
            pl.BlockSpec((_RB, HID), lambda i: (i, 0)),
            pl.BlockSpec((_RB, HID), lambda i: (i, 0)),
            pl.BlockSpec((_RB, HID), lambda i: (i, 0)),
            pl.BlockSpec((_RB, 1), lambda i: (i, 0)),
            pl.BlockSpec((_RB, 1), lambda i: (i, 0)),
            pl.BlockSpec((1, HID), lambda i: (0, 0)),
            pl.BlockSpec((1, 1, _RB), lambda i: (i, 0, 0)),
        ],
        out_specs=pl.BlockSpec((NG, HID), lambda i: (0, 0)),
        out_shape=_f32((NG, HID)),
        scratch_shapes=[pltpu.VMEM((NG, HID), jnp.float32)],
    )(acca, accb, y2, dega, degb, b2, batch3)


def kernel(x, mapping, edge_index, edge_attr, batch, emb, W1, b1, W2, b2):
    src = edge_index[0]
    dst = edge_index[1]
    map_pad = jnp.concatenate([mapping, jnp.zeros((NNP - NN,), jnp.int32)])
    emb2 = emb.reshape(emb.shape[0] // 2, 2 * EMB)

    nemb2, deg = _sc_prep(map_pad, dst, edge_attr, emb2)
    dega = deg[:NN].reshape(NN, 1)
    degb = deg[NNP:NNP + NN].reshape(NN, 1)

    y1 = _tc_mm1(x, nemb2[:NN], mapping.reshape(NN, 1), dega, degb,
                 W1[:FEAT], W1[FEAT:])

    acc1 = _sc_mp(y1, src, dst, edge_attr)
    y2 = _tc_mm2(acc1[:NN], acc1[NNP:NNP + NN], y1, dega, degb,
                 b1.reshape(1, HID), W2)

    acc2 = _sc_mp(y2, src, dst, edge_attr)
    out = _tc_pool(acc2[:NN], acc2[NNP:NNP + NN], y2, dega, degb,
                   b2.reshape(1, HID), batch.reshape(_NRB, 1, _RB))
    return out


# speedup vs baseline: 15.7859x; 1.1025x over previous
import jax
import jax.numpy as jnp
from jax import lax
from jax.experimental import pallas as pl
from jax.experimental.pallas import tpu as pltpu
from jax.experimental.pallas import tpu_sc as plsc

NN = 10000
NNP = 10240
NE = 320000
FEAT = 128
EMB = 64
HID = 128
NG = 32
EW = 128
N_EWIN = NE // EW
GW = 128
N_GWIN = NNP // GW
NC = 2
NS = 16
NWORK = NC * NS
DPT = NNP // NS

_HIGH = lax.Precision.HIGHEST


def _f32(shape):
    return jax.ShapeDtypeStruct(shape, jnp.float32)


def _sc_prep_body(map_hbm, e_hbm, attr_hbm, emb2_hbm,
                  nemb_hbm, deg_hbm,
                  mapv, mapv2, rows, dsti, attrv, zbuf, deg_sh, sem):
    c = lax.axis_index("c")
    s = lax.axis_index("s")
    wid = c * NS + s

    z16 = jnp.zeros((16,), jnp.float32)

    def zb(i, carry):
        zbuf[pl.ds(i * 16, 16)] = z16
        return carry

    lax.fori_loop(0, DPT // 16, zb, 0)
    pltpu.sync_copy(zbuf, deg_sh.at[pl.ds(s * DPT, DPT)])
    plsc.subcore_barrier()

    n_e = (N_EWIN - wid + NWORK - 1) // NWORK

    def ebody(k, carry):
        base = (wid + k * NWORK) * EW
        pltpu.sync_copy(e_hbm.at[pl.ds(NE + base, EW)], dsti.at[0])
        pltpu.sync_copy(attr_hbm.at[pl.ds(base, EW)], attrv)
        pltpu.sync_copy(attrv, deg_sh.at[dsti.at[0]], add=True)
        return carry

    lax.fori_loop(0, n_e, ebody, 0)

    n_g = (N_GWIN - wid + NWORK - 1) // NWORK

    def gbody(k, carry):
        base = (wid + k * NWORK) * GW
        pltpu.sync_copy(map_hbm.at[pl.ds(base, GW)], mapv)
        for j in range(GW // 16):
            sl = pl.ds(j * 16, 16)
            mapv2[sl] = mapv[sl] >> 1
        pltpu.async_copy(emb2_hbm.at[mapv2], rows, sem).wait()
        pltpu.sync_copy(rows, nemb_hbm.at[pl.ds(base, GW)])
        return carry

    lax.fori_loop(0, n_g, gbody, 0)

    plsc.subcore_barrier()

    pltpu.sync_copy(deg_sh.at[pl.ds(s * DPT, DPT)], zbuf)
    pltpu.sync_copy(zbuf, deg_hbm.at[pl.ds(c * NNP + s * DPT, DPT)])


def _sc_prep(map_pad, eflat, attr, emb2):
    mesh = plsc.VectorSubcoreMesh(core_axis_name="c", subcore_axis_name="s")
    f = pl.kernel(
        _sc_prep_body,
        out_type=(_f32((NNP, FEAT)), _f32((2 * NNP,))),
        mesh=mesh,
        scratch_types=[
            pltpu.VMEM((GW,), jnp.int32),
            pltpu.VMEM((GW,), jnp.int32),
            pltpu.VMEM((GW, FEAT), jnp.float32),
            pltpu.VMEM((1, EW), jnp.int32),
            pltpu.VMEM((EW,), jnp.float32),
            pltpu.VMEM((DPT,), jnp.float32),
            pltpu.VMEM_SHARED((NNP,), jnp.float32),
            pltpu.SemaphoreType.DMA,
        ],
        compiler_params=pltpu.CompilerParams(needs_layout_passes=False),
    )
    return f(map_pad, eflat, attr, emb2)


MW = 64
N_MWIN = NE // MW
WPW = 156


def _sc_mp_body(y_hbm, e_hbm, attr_hbm,
                out_hbm,
                ra, rb, rc, sia, sib, sic, dxa, dxb, dxc, wva, wvb, wvc,
                acc_sh,
                sga, sgb, sgc, ssa, ssb, ssc, sla, slb, slc):
    c = lax.axis_index("c")
    s = lax.axis_index("s")
    wid = c * NS + s
    z16 = jnp.zeros((16,), jnp.float32)
    bufs = ((ra, sia, dxa, wva, sga, ssa, sla),
            (rb, sib, dxb, wvb, sgb, ssb, slb),
            (rc, sic, dxc, wvc, sgc, ssc, slc))
    wbase = wid * WPW

    def zrow(i, carry):
        for j in range(HID // 16):
            ra[i, pl.ds(j * 16, 16)] = z16
        return carry

    lax.fori_loop(0, MW, zrow, 0)
    for cb in range(0, DPT, MW):
        pltpu.sync_copy(ra, acc_sh.at[pl.ds(s * DPT + cb, MW)])
    plsc.subcore_barrier()

    def scale(b):
        rows, wv = bufs[b][0], bufs[b][3]

        @plsc.parallel_loop(0, MW, 1, unroll=4)
        def _(i):
            ws = plsc.load_gather(wv, [jnp.full((16,), i, dtype=jnp.int32)])
            for j in range(HID // 16):
                sl = pl.ds(j * 16, 16)
                rows[i, sl] = rows[i, sl] * ws

    def start_idx_loads(b, k):
        _, si, dx, wv, _, _, sl = bufs[b]
        ebase = (wbase + k) * MW
        pltpu.async_copy(e_hbm.at[pl.ds(ebase, MW)], si, sl)
        pltpu.async_copy(e_hbm.at[pl.ds(NE + ebase, MW)], dx.at[0], sl)
        pltpu.async_copy(attr_hbm.at[pl.ds(ebase, MW)], wv, sl)

    def wait_idx_loads(b, k):
        _, si, dx, wv, _, _, sl = bufs[b]
        ebase = (wbase + k) * MW
        pltpu.make_async_copy(e_hbm.at[pl.ds(ebase, MW)], si, sl).wait()
        pltpu.make_async_copy(e_hbm.at[pl.ds(NE + ebase, MW)], dx.at[0], sl).wait()
        pltpu.make_async_copy(attr_hbm.at[pl.ds(ebase, MW)], wv, sl).wait()

    def start_gather(b):
        rows, si, sg = bufs[b][0], bufs[b][1], bufs[b][4]
        pltpu.async_copy(y_hbm.at[si], rows, sg)

    def wait_gather(b):
        rows, si, sg = bufs[b][0], bufs[b][1], bufs[b][4]
        pltpu.make_async_copy(y_hbm.at[si], rows, sg).wait()

    def start_scatter(b):
        rows, dx, ss = bufs[b][0], bufs[b][2], bufs[b][5]
        pltpu.async_copy(rows, acc_sh.at[dx.at[0]], ss, add=True)

    def wait_scatter(b):
        rows, dx, ss = bufs[b][0], bufs[b][2], bufs[b][5]
        pltpu.make_async_copy(rows, acc_sh.at[dx.at[0]], ss).wait()

    start_idx_loads(0, 0)
    start_idx_loads(1, 1)
    wait_idx_loads(0, 0)
    start_gather(0)

    def pbody(p, carry):
        for j in range(3):
            k = 3 * p + j
            b, bB, bA = j, (j + 1) % 3, (j + 2) % 3

            @pl.when(k + 1 < WPW)
            def _():
                wait_idx_loads(bB, k + 1)
                start_gather(bB)

            wait_gather(b)
            scale(b)
            start_scatter(b)

            @pl.when(k + 2 < WPW)
            def _():
                @pl.when(k >= 1)
                def _():
                    wait_scatter(bA)

                start_idx_loads(bA, k + 2)
        return carry

    lax.fori_loop(0, WPW // 3, pbody, 0)
    for b in range(3):
        wait_scatter(b)

    @pl.when(wid < N_MWIN - NWORK * WPW)
    def _():
        lbase = (NWORK * WPW + wid) * MW
        pltpu.sync_copy(e_hbm.at[pl.ds(lbase, MW)], sia)
        pltpu.sync_copy(e_hbm.at[pl.ds(NE + lbase, MW)], dxa.at[0])
        pltpu.sync_copy(attr_hbm.at[pl.ds(lbase, MW)], wva)
        start_gather(0)
        wait_gather(0)
        scale(0)
        pltpu.sync_copy(ra, acc_sh.at[dxa.at[0]], add=True)

    plsc.subcore_barrier()
    for cb in range(0, DPT, MW):
        pltpu.sync_copy(acc_sh.at[pl.ds(s * DPT + cb, MW)], ra)
        pltpu.sync_copy(ra, out_hbm.at[pl.ds(c * NNP + s * DPT + cb, MW)])


def _sc_mp(y, eflat, attr):
    mesh = plsc.VectorSubcoreMesh(core_axis_name="c", subcore_axis_name="s")
    f = pl.kernel(
        _sc_mp_body,
        out_type=_f32((2 * NNP, HID)),
        mesh=mesh,
        scratch_types=[
            pltpu.VMEM((MW, HID), jnp.float32),
            pltpu.VMEM((MW, HID), jnp.float32),
            pltpu.VMEM((MW, HID), jnp.float32),
            pltpu.VMEM((MW,), jnp.int32),
            pltpu.VMEM((MW,), jnp.int32),
            pltpu.VMEM((MW,), jnp.int32),
            pltpu.VMEM((1, MW), jnp.int32),
            pltpu.VMEM((1, MW), jnp.int32),
            pltpu.VMEM((1, MW), jnp.int32),
            pltpu.VMEM((MW,), jnp.float32),
            pltpu.VMEM((MW,), jnp.float32),
            pltpu.VMEM((MW,), jnp.float32),
            pltpu.VMEM_SHARED((NNP, HID), jnp.float32),
            pltpu.SemaphoreType.DMA,
            pltpu.SemaphoreType.DMA,
            pltpu.SemaphoreType.DMA,
            pltpu.SemaphoreType.DMA,
            pltpu.SemaphoreType.DMA,
            pltpu.SemaphoreType.DMA,
            pltpu.SemaphoreType.DMA,
            pltpu.SemaphoreType.DMA,
            pltpu.SemaphoreType.DMA,
        ],
        compiler_params=pltpu.CompilerParams(needs_layout_passes=False),
    )
    return f(y, eflat, attr)


_RB = 1000
_NRB = NN // _RB


def _dis(da, db):
    deg = da + db + 1.0
    return jnp.where(deg > 0, lax.rsqrt(jnp.maximum(deg, 1e-12)), 0.0)


def _mm(a, b):
    return lax.dot_general(a, b, (((1,), (0,)), ((), ())),
                           precision=_HIGH, preferred_element_type=jnp.float32)


def _mm1_body(x_ref, ne_ref, m_ref, da_ref, db_ref, w1x_ref, w1e_ref, y_ref):
    dis = _dis(da_ref[...], db_ref[...])
    par = (m_ref[...] & 1) == 1
    ne = ne_ref[...]
    e = jnp.where(par, ne[:, EMB:], ne[:, :EMB])
    xw = _mm(x_ref[...], w1x_ref[...]) + _mm(e, w1e_ref[...])
    y_ref[...] = dis * xw


def _tc_mm1(x, nemb2, map_col, dega, degb, w1x, w1e):
    return pl.pallas_call(
        _mm1_body,
        grid=(_NRB,),
        in_specs=[
            pl.BlockSpec((_RB, FEAT), lambda i: (i, 0)),
            pl.BlockSpec((_RB, FEAT), lambda i: (i, 0)),
            pl.BlockSpec((_RB, 1), lambda i: (i, 0)),
            pl.BlockSpec((_RB, 1), lambda i: (i, 0)),
            pl.BlockSpec((_RB, 1), lambda i: (i, 0)),
            pl.BlockSpec((FEAT, HID), lambda i: (0, 0)),
            pl.BlockSpec((EMB, HID), lambda i: (0, 0)),
        ],
        out_specs=pl.BlockSpec((_RB, HID), lambda i: (i, 0)),
        out_shape=_f32((NN, HID)),
    )(x, nemb2, map_col, dega, degb, w1x, w1e)


def _mm2_body(aa_ref, ab_ref, y1_ref, da_ref, db_ref, b1_ref, w2_ref, y2_ref):
    dis = _dis(da_ref[...], db_ref[...])
    h1 = jnp.maximum(dis * (aa_ref[...] + ab_ref[...] + y1_ref[...]) + b1_ref[...], 0.0)
    y2_ref[...] = dis * _mm(h1, w2_ref[...])


def _tc_mm2(acca, accb, y1, dega, degb, b1, w2):
    return pl.pallas_call(
        _mm2_body,
        grid=(_NRB,),
        in_specs=[
            pl.BlockSpec((_RB, HID), lambda i: (i, 0)),
            pl.BlockSpec((_RB, HID), lambda i: (i, 0)),
            pl.BlockSpec((_RB, HID), lambda i: (i, 0)),
            pl.BlockSpec((_RB, 1), lambda i: (i, 0)),
            pl.BlockSpec((_RB, 1), lambda i: (i, 0)),
            pl.BlockSpec((1, HID), lambda i: (0, 0)),
            pl.BlockSpec((HID, HID), lambda i: (0, 0)),
        ],
        out_specs=pl.BlockSpec((_RB, HID), lambda i: (i, 0)),
        out_shape=_f32((NN, HID)),
    )(acca, accb, y1, dega, degb, b1, w2)


def _pool_body(aa_ref, ab_ref, y2_ref, da_ref, db_ref, b2_ref, batch_ref,
               out_ref, cnt_ref):
    i = pl.program_id(0)

    @pl.when(i == 0)
    def _():
        out_ref[...] = jnp.zeros_like(out_ref)
        cnt_ref[...] = jnp.zeros_like(cnt_ref)

    dis = _dis(da_ref[...], db_ref[...])
    h2 = jnp.maximum(dis * (aa_ref[...] + ab_ref[...] + y2_ref[...]) + b2_ref[...], 0.0)
    b = batch_ref[0]
    gi = lax.broadcasted_iota(jnp.int32, (NG, _RB), 0)
    onehot = (gi == jnp.broadcast_to(b, (NG, _RB))).astype(jnp.float32)
    out_ref[...] += _mm(onehot, h2)
    cnt_ref[...] += jnp.broadcast_to(
        jnp.sum(onehot, axis=1, keepdims=True), (NG, HID))

    @pl.when(i == _NRB - 1)
    def _():
        out_ref[...] = out_ref[...] / jnp.maximum(cnt_ref[...], 1.0)


def _tc_pool(acca, accb, y2, dega, degb, b2, batch3):
    return pl.pallas_call(
        _pool_body,
        grid=(_NRB,),
        in_specs=[
            pl.BlockSpec((_RB, HID), lambda i: (i, 0)),
            pl.BlockSpec((_RB, HID), lambda i: (i, 0)),
            pl.BlockSpec((_RB, HID), lambda i: (i, 0)),
            pl.BlockSpec((_RB, 1), lambda i: (i, 0)),
            pl.BlockSpec((_RB, 1), lambda i: (i, 0)),
            pl.BlockSpec((1, HID), lambda i: (0, 0)),
            pl.BlockSpec((1, 1, _RB), lambda i: (i, 0, 0)),
        ],
        out_specs=pl.BlockSpec((NG, HID), lambda i: (0, 0)),
        out_shape=_f32((NG, HID)),
        scratch_shapes=[pltpu.VMEM((NG, HID), jnp.float32)],
    )(acca, accb, y2, dega, degb, b2, batch3)


def kernel(x, mapping, edge_index, edge_attr, batch, emb, W1, b1, W2, b2):
    eflat = edge_index.reshape(2 * NE)
    map_pad = jnp.concatenate([mapping, jnp.zeros((NNP - NN,), jnp.int32)])
    emb2 = emb.reshape(emb.shape[0] // 2, 2 * EMB)

    nemb2, deg = _sc_prep(map_pad, eflat, edge_attr, emb2)
    dega = deg[:NN].reshape(NN, 1)
    degb = deg[NNP:NNP + NN].reshape(NN, 1)

    y1 = _tc_mm1(x, nemb2[:NN], mapping.reshape(NN, 1), dega, degb,
                 W1[:FEAT], W1[FEAT:])

    acc1 = _sc_mp(y1, eflat, edge_attr)
    y2 = _tc_mm2(acc1[:NN], acc1[NNP:NNP + NN], y1, dega, degb,
                 b1.reshape(1, HID), W2)

    acc2 = _sc_mp(y2, eflat, edge_attr)
    out = _tc_pool(acc2[:NN], acc2[NNP:NNP + NN], y2, dega, degb,
                   b2.reshape(1, HID), batch.reshape(_NRB, 1, _RB))
    return out
